# baseline (speedup 1.0000x reference)
import functools

import jax
import jax.numpy as jnp
from jax.experimental import pallas as pl
from jax.experimental.pallas import tpu as pltpu

NEG_SLOPE = 0.01
IN_EPS = 1e-5
ACT_DTYPE = jnp.bfloat16
W_DTYPE = jnp.bfloat16

_PARAMS = pltpu.CompilerParams(
    dimension_semantics=("parallel",),
    vmem_limit_bytes=48 * 1024 * 1024,
)


def _leaky(x):
    return jnp.maximum(x, NEG_SLOPE * x)


def _pool2x2(x, nb, h, w):
    c = x.shape[-1]
    x = jnp.max(x.reshape(nb, h, 2, 2 * w, c), axis=2)
    return jnp.max(x.reshape(nb, h, w, 2, c), axis=3)


def _conv3x3(x, w3_ref, nb):
    _, h, w, c = x.shape
    n = w3_ref.shape[-1] // 3
    xb = x.astype(W_DTYPE)
    zr = jnp.zeros((nb, 1, w, c), W_DTYPE)
    xp = jnp.concatenate([zr, xb, zr], axis=1)
    cols = jnp.concatenate(
        [xp[:, ky:ky + h] for ky in range(3)], axis=-1)
    cols = cols.reshape(nb * h * w, 3 * c)
    p = jnp.dot(cols, w3_ref[...], preferred_element_type=jnp.float32)
    p0, p1, p2 = p[:, :n], p[:, n:2 * n], p[:, 2 * n:]
    m = nb * h * w
    xcol = jax.lax.broadcasted_iota(jnp.int32, (m, 1), 0) & (w - 1)
    zrow = jnp.zeros((1, n), jnp.float32)
    pd = jnp.concatenate([zrow, p0[:-1]], axis=0)
    pd = jnp.where(xcol == 0, 0.0, pd)
    pu = jnp.concatenate([p2[1:], zrow], axis=0)
    pu = jnp.where(xcol == w - 1, 0.0, pu)
    return p1 + pd + pu


def _conv3x3_in(x, w3_ref, nb):
    acc = _conv3x3(x, w3_ref, nb)
    a = acc.reshape(nb, -1, acc.shape[-1])
    mean = jnp.mean(a, axis=1, keepdims=True)
    msq = jnp.mean(jnp.square(a), axis=1, keepdims=True)
    var = msq - jnp.square(mean)
    s = jax.lax.rsqrt(var + IN_EPS)
    a = a * s + (-mean * s)
    return _leaky(a).reshape(acc.shape)


def _conv3x3_b(x, w3_ref, b_ref, nb):
    return _leaky(_conv3x3(x, w3_ref, nb) + b_ref[...])


def _conv_t2x2(a, wu_ref, bu_ref, nb, h, w):
    cup = wu_ref.shape[-1] // 4
    y = jnp.dot(a, wu_ref[...], preferred_element_type=jnp.float32) + bu_ref[...]
    ys = [y[:, k * cup:(k + 1) * cup].reshape(nb, h, w, cup) for k in range(4)]
    row0 = jnp.stack([ys[0], ys[1]], axis=3).reshape(nb, h, 2 * w, cup)
    row1 = jnp.stack([ys[2], ys[3]], axis=3).reshape(nb, h, 2 * w, cup)
    return jnp.stack([row0, row1], axis=2).reshape(nb, 2 * h, 2 * w, cup)


def _down_kernel(x_ref, w1_ref, w2_ref, o_ref, *, h, w, pool, nb):
    x = x_ref[...].astype(jnp.float32)
    if pool:
        x = _pool2x2(x, nb, h, w)
    a = _conv3x3_in(x, w1_ref, nb)
    a = _conv3x3_in(a.reshape(nb, h, w, -1), w2_ref, nb)
    o_ref[...] = a.reshape(nb, h, w, -1).astype(o_ref.dtype)


def _mid_kernel(x_ref, w1_ref, w2_ref, wu_ref, bu_ref, o_ref, *, h, w, nb):
    x = x_ref[...].astype(jnp.float32)
    p = _pool2x2(x, nb, h, w)
    a = _conv3x3_in(p, w1_ref, nb)
    a = _conv3x3_in(a.reshape(nb, h, w, -1), w2_ref, nb)
    up = _conv_t2x2(a.astype(W_DTYPE), wu_ref, bu_ref, nb, h, w)
    o_ref[...] = jnp.concatenate([x, up], axis=-1).astype(o_ref.dtype)


def _up_kernel(sub_ref, skip_ref, w3_ref, b3_ref, w4_ref, b4_ref,
               wu_ref, bu_ref, o_ref, *, h, w, nb):
    a = _conv3x3_b(sub_ref[...].astype(jnp.float32), w3_ref, b3_ref, nb)
    a = _conv3x3_b(a.reshape(nb, h, w, -1), w4_ref, b4_ref, nb)
    up = _conv_t2x2(a.astype(W_DTYPE), wu_ref, bu_ref, nb, h, w)
    o_ref[...] = jnp.concatenate(
        [skip_ref[...].astype(jnp.float32), up], axis=-1).astype(o_ref.dtype)


def _head_kernel(sub_ref, w3_ref, b3_ref, w4_ref, b4_ref, wf_ref, bf_ref,
                 ls_ref, sm_ref, *, h, w, nb):
    a = _conv3x3_b(sub_ref[...].astype(jnp.float32), w3_ref, b3_ref, nb)
    a = _conv3x3_b(a.reshape(nb, h, w, -1), w4_ref, b4_ref, nb)
    nc = wf_ref.shape[-1]
    logits = jnp.dot(a.astype(W_DTYPE), wf_ref[...],
                     preferred_element_type=jnp.float32) + bf_ref[...]
    m = jnp.max(logits, axis=-1, keepdims=True)
    z = logits - m
    e = jnp.exp(z)
    s = jnp.sum(e, axis=-1, keepdims=True)
    sm_ref[...] = (e / s).reshape(nb, h, w, nc)
    ls_ref[...] = (z - jnp.log(s)).reshape(nb, h, w, nc)


def _pack_w3(w):
    nine_c, n = w.shape
    c = nine_c // 9
    w4 = jnp.transpose(w.reshape(3, 3, c, n), (1, 0, 2, 3))
    return jnp.transpose(w4.reshape(3, 3 * c, n), (1, 0, 2)).reshape(3 * c, 3 * n)


def _pack_uw(wu, bu):
    cin, cup = wu.shape[1], wu.shape[2]
    return (jnp.transpose(wu, (1, 0, 2)).reshape(cin, 4 * cup),
            jnp.tile(bu, (1, 4)))


def _const_spec(a):
    nd = a.ndim
    return pl.BlockSpec(a.shape, lambda n: (0,) * nd)


def _batch_spec(shape):
    return pl.BlockSpec(shape, lambda n: (n, 0, 0, 0))


def down_block(x, w1, w2, *, pool, nb):
    N, H, W, Cin = x.shape
    h, w = (H // 2, W // 2) if pool else (H, W)
    Cout = w2.shape[-1] // 3
    kern = functools.partial(_down_kernel, h=h, w=w, pool=pool, nb=nb)
    return pl.pallas_call(
        kern,
        out_shape=jax.ShapeDtypeStruct((N, h, w, Cout), ACT_DTYPE),
        grid=(N // nb,),
        in_specs=[_batch_spec((nb, H, W, Cin)), _const_spec(w1), _const_spec(w2)],
        out_specs=_batch_spec((nb, h, w, Cout)),
        compiler_params=_PARAMS,
    )(x, w1, w2)


def mid_block(x, w1, w2, wu, bu, *, nb):
    N, H, W, Cin = x.shape
    h, w = H // 2, W // 2
    Cup = wu.shape[-1] // 4
    kern = functools.partial(_mid_kernel, h=h, w=w, nb=nb)
    return pl.pallas_call(
        kern,
        out_shape=jax.ShapeDtypeStruct((N, H, W, Cin + Cup), ACT_DTYPE),
        grid=(N // nb,),
        in_specs=[_batch_spec((nb, H, W, Cin)), _const_spec(w1),
                  _const_spec(w2), _const_spec(wu), _const_spec(bu)],
        out_specs=_batch_spec((nb, H, W, Cin + Cup)),
        compiler_params=_PARAMS,
    )(x, w1, w2, wu, bu)


def up_block(sub, skip, w3, b3, w4, b4, wu, bu, *, nb):
    N, h, w, Csub = sub.shape
    _, H, W, Cs = skip.shape
    Cup = wu.shape[-1] // 4
    kern = functools.partial(_up_kernel, h=h, w=w, nb=nb)
    return pl.pallas_call(
        kern,
        out_shape=jax.ShapeDtypeStruct((N, H, W, Cs + Cup), ACT_DTYPE),
        grid=(N // nb,),
        in_specs=[_batch_spec((nb, h, w, Csub)), _batch_spec((nb, H, W, Cs)),
                  _const_spec(w3), _const_spec(b3), _const_spec(w4),
                  _const_spec(b4), _const_spec(wu), _const_spec(bu)],
        out_specs=_batch_spec((nb, H, W, Cs + Cup)),
        compiler_params=_PARAMS,
    )(sub, skip, w3, b3, w4, b4, wu, bu)


def head_block(sub, w3, b3, w4, b4, wf, bf_, *, nb):
    N, H, W, Csub = sub.shape
    nc = wf.shape[-1]
    kern = functools.partial(_head_kernel, h=H, w=W, nb=nb)
    out_spec = _batch_spec((nb, H, W, nc))
    return pl.pallas_call(
        kern,
        out_shape=(jax.ShapeDtypeStruct((N, H, W, nc), jnp.float32),
                   jax.ShapeDtypeStruct((N, H, W, nc), jnp.float32)),
        grid=(N // nb,),
        in_specs=[_batch_spec((nb, H, W, Csub)), _const_spec(w3),
                  _const_spec(b3), _const_spec(w4), _const_spec(b4),
                  _const_spec(wf), _const_spec(bf_)],
        out_specs=(out_spec, out_spec),
        compiler_params=_PARAMS,
    )(sub, w3, b3, w4, b4, wf, bf_)


def kernel(x, o_c1w, o_c1b, o_c2w, o_c2b, o_c3w, o_c3b, o_c4w, o_c4b,
           o_fw, o_fb,
           d3_c1w, d3_c1b, d3_c2w, d3_c2b, d3_c3w, d3_c3b, d3_c4w, d3_c4b,
           d3_uw, d3_ub,
           d2_c1w, d2_c1b, d2_c2w, d2_c2b, d2_c3w, d2_c3b, d2_c4w, d2_c4b,
           d2_uw, d2_ub,
           d1_c1w, d1_c1b, d1_c2w, d1_c2b, d1_c3w, d1_c3b, d1_c4w, d1_c4b,
           d1_uw, d1_ub,
           m_c1w, m_c1b, m_c2w, m_c2b, m_uw, m_ub):
    xh = jnp.transpose(x, (0, 2, 3, 1)).astype(ACT_DTYPE)
    m_uw, m_ub = _pack_uw(m_uw, m_ub)
    d1_uw, d1_ub = _pack_uw(d1_uw, d1_ub)
    d2_uw, d2_ub = _pack_uw(d2_uw, d2_ub)
    d3_uw, d3_ub = _pack_uw(d3_uw, d3_ub)
    (o_c1w, o_c2w, o_c3w, o_c4w, d3_c1w, d3_c2w, d3_c3w, d3_c4w,
     d2_c1w, d2_c2w, d2_c3w, d2_c4w, d1_c1w, d1_c2w, d1_c3w, d1_c4w,
     m_c1w, m_c2w) = [
        _pack_w3(w) for w in
        (o_c1w, o_c2w, o_c3w, o_c4w, d3_c1w, d3_c2w, d3_c3w, d3_c4w,
         d2_c1w, d2_c2w, d2_c3w, d2_c4w, d1_c1w, d1_c2w, d1_c3w, d1_c4w,
         m_c1w, m_c2w)]

    h0 = down_block(xh, o_c1w, o_c2w, pool=False, nb=1)
    h1 = down_block(h0, d3_c1w, d3_c2w, pool=True, nb=1)
    h2 = down_block(h1, d2_c1w, d2_c2w, pool=True, nb=2)
    h3 = down_block(h2, d1_c1w, d1_c2w, pool=True, nb=4)

    u3 = mid_block(h3, m_c1w, m_c2w, m_uw, m_ub, nb=8)
    u2 = up_block(u3, h2, d1_c3w, d1_c3b, d1_c4w, d1_c4b,
                  d1_uw, d1_ub, nb=4)
    u1 = up_block(u2, h1, d2_c3w, d2_c3b, d2_c4w, d2_c4b,
                  d2_uw, d2_ub, nb=2)
    u0 = up_block(u1, h0, d3_c3w, d3_c3b, d3_c4w, d3_c4b,
                  d3_uw, d3_ub, nb=1)

    ls, sm = head_block(u0, o_c3w, o_c3b, o_c4w, o_c4b, o_fw, o_fb, nb=1)
    return {'log_softmax': jnp.transpose(ls, (0, 3, 1, 2)),
            'softmax': jnp.transpose(sm, (0, 3, 1, 2))}


# speedup vs baseline: 1.3150x; 1.3150x over previous
import functools

import jax
import jax.numpy as jnp
from jax.experimental import pallas as pl
from jax.experimental.pallas import tpu as pltpu

NEG_SLOPE = 0.01
IN_EPS = 1e-5
ACT_DTYPE = jnp.bfloat16
W_DTYPE = jnp.bfloat16

_PARAMS = pltpu.CompilerParams(
    dimension_semantics=("parallel",),
    vmem_limit_bytes=48 * 1024 * 1024,
)


def _leaky(x):
    return jnp.maximum(x, NEG_SLOPE * x)


def _pool2x2(x, nb, h, w):
    c = x.shape[-1]
    x = jnp.max(x.reshape(nb, h, 2, 2 * w, c), axis=2)
    return jnp.max(x.reshape(nb, h, w, 2, c), axis=3)


def _conv3x3(x, w3_ref, nb):
    _, h, w, c = x.shape
    n = w3_ref.shape[-1]
    xb = x.astype(W_DTYPE)
    zr = jnp.zeros((nb, 1, w, c), W_DTYPE)
    xp = jnp.concatenate([zr, xb, zr], axis=1)
    cols = jnp.concatenate(
        [xp[:, ky:ky + h] for ky in range(3)], axis=-1)
    cols = cols.reshape(nb * h * w, 3 * c)
    p0 = jnp.dot(cols, w3_ref[0], preferred_element_type=jnp.float32)
    p1 = jnp.dot(cols, w3_ref[1], preferred_element_type=jnp.float32)
    p2 = jnp.dot(cols, w3_ref[2], preferred_element_type=jnp.float32)
    m = nb * h * w
    xcol = jax.lax.broadcasted_iota(jnp.int32, (m, 1), 0) & (w - 1)
    zrow = jnp.zeros((1, n), jnp.float32)
    pd = jnp.concatenate([zrow, p0[:-1]], axis=0)
    pd = jnp.where(xcol == 0, 0.0, pd)
    pu = jnp.concatenate([p2[1:], zrow], axis=0)
    pu = jnp.where(xcol == w - 1, 0.0, pu)
    return p1 + pd + pu


def _conv3x3_in(x, w3_ref, nb):
    acc = _conv3x3(x, w3_ref, nb)
    a = acc.reshape(nb, -1, acc.shape[-1])
    mean = jnp.mean(a, axis=1, keepdims=True)
    msq = jnp.mean(jnp.square(a), axis=1, keepdims=True)
    var = msq - jnp.square(mean)
    s = jax.lax.rsqrt(var + IN_EPS)
    a = a * s + (-mean * s)
    return _leaky(a).reshape(acc.shape)


def _conv3x3_b(x, w3_ref, b_ref, nb):
    return _leaky(_conv3x3(x, w3_ref, nb) + b_ref[...])


def _conv_t2x2(a, wu_ref, bu_ref, nb, h, w):
    cup = wu_ref.shape[-1] // 4
    y = jnp.dot(a, wu_ref[...], preferred_element_type=jnp.float32) + bu_ref[...]
    ys = [y[:, k * cup:(k + 1) * cup].reshape(nb, h, w, cup) for k in range(4)]
    row0 = jnp.stack([ys[0], ys[1]], axis=3).reshape(nb, h, 2 * w, cup)
    row1 = jnp.stack([ys[2], ys[3]], axis=3).reshape(nb, h, 2 * w, cup)
    return jnp.stack([row0, row1], axis=2).reshape(nb, 2 * h, 2 * w, cup)


def _down_kernel(x_ref, w1_ref, w2_ref, o_ref, *, h, w, pool, nb):
    x = x_ref[...].astype(jnp.float32)
    if pool:
        x = _pool2x2(x, nb, h, w)
    a = _conv3x3_in(x, w1_ref, nb)
    a = _conv3x3_in(a.reshape(nb, h, w, -1), w2_ref, nb)
    o_ref[...] = a.reshape(nb, h, w, -1).astype(o_ref.dtype)


def _mid_kernel(x_ref, w1_ref, w2_ref, wu_ref, bu_ref, o_ref, *, h, w, nb):
    x = x_ref[...].astype(jnp.float32)
    p = _pool2x2(x, nb, h, w)
    a = _conv3x3_in(p, w1_ref, nb)
    a = _conv3x3_in(a.reshape(nb, h, w, -1), w2_ref, nb)
    up = _conv_t2x2(a.astype(W_DTYPE), wu_ref, bu_ref, nb, h, w)
    o_ref[...] = jnp.concatenate([x, up], axis=-1).astype(o_ref.dtype)


def _up_kernel(sub_ref, skip_ref, w3_ref, b3_ref, w4_ref, b4_ref,
               wu_ref, bu_ref, o_ref, *, h, w, nb):
    a = _conv3x3_b(sub_ref[...].astype(jnp.float32), w3_ref, b3_ref, nb)
    a = _conv3x3_b(a.reshape(nb, h, w, -1), w4_ref, b4_ref, nb)
    up = _conv_t2x2(a.astype(W_DTYPE), wu_ref, bu_ref, nb, h, w)
    o_ref[...] = jnp.concatenate(
        [skip_ref[...].astype(jnp.float32), up], axis=-1).astype(o_ref.dtype)


def _head_kernel(sub_ref, w3_ref, b3_ref, w4_ref, b4_ref, wf_ref, bf_ref,
                 ls_ref, sm_ref, *, h, w, nb):
    a = _conv3x3_b(sub_ref[...].astype(jnp.float32), w3_ref, b3_ref, nb)
    a = _conv3x3_b(a.reshape(nb, h, w, -1), w4_ref, b4_ref, nb)
    nc = wf_ref.shape[-1]
    logits = jnp.dot(a.astype(W_DTYPE), wf_ref[...],
                     preferred_element_type=jnp.float32) + bf_ref[...]
    m = jnp.max(logits, axis=-1, keepdims=True)
    z = logits - m
    e = jnp.exp(z)
    s = jnp.sum(e, axis=-1, keepdims=True)
    sm_ref[...] = (e / s).reshape(nb, h, w, nc)
    ls_ref[...] = (z - jnp.log(s)).reshape(nb, h, w, nc)


def _pack_w3(w):
    nine_c, n = w.shape
    c = nine_c // 9
    return jnp.transpose(w.reshape(3, 3, c, n), (1, 0, 2, 3)).reshape(3, 3 * c, n)


def _pack_uw(wu, bu):
    cin, cup = wu.shape[1], wu.shape[2]
    return (jnp.transpose(wu, (1, 0, 2)).reshape(cin, 4 * cup),
            jnp.tile(bu, (1, 4)))


def _const_spec(a):
    nd = a.ndim
    return pl.BlockSpec(a.shape, lambda n: (0,) * nd)


def _batch_spec(shape):
    return pl.BlockSpec(shape, lambda n: (n, 0, 0, 0))


def down_block(x, w1, w2, *, pool, nb):
    N, H, W, Cin = x.shape
    h, w = (H // 2, W // 2) if pool else (H, W)
    Cout = w2.shape[-1]
    kern = functools.partial(_down_kernel, h=h, w=w, pool=pool, nb=nb)
    return pl.pallas_call(
        kern,
        out_shape=jax.ShapeDtypeStruct((N, h, w, Cout), ACT_DTYPE),
        grid=(N // nb,),
        in_specs=[_batch_spec((nb, H, W, Cin)), _const_spec(w1), _const_spec(w2)],
        out_specs=_batch_spec((nb, h, w, Cout)),
        compiler_params=_PARAMS,
    )(x, w1, w2)


def mid_block(x, w1, w2, wu, bu, *, nb):
    N, H, W, Cin = x.shape
    h, w = H // 2, W // 2
    Cup = wu.shape[-1] // 4
    kern = functools.partial(_mid_kernel, h=h, w=w, nb=nb)
    return pl.pallas_call(
        kern,
        out_shape=jax.ShapeDtypeStruct((N, H, W, Cin + Cup), ACT_DTYPE),
        grid=(N // nb,),
        in_specs=[_batch_spec((nb, H, W, Cin)), _const_spec(w1),
                  _const_spec(w2), _const_spec(wu), _const_spec(bu)],
        out_specs=_batch_spec((nb, H, W, Cin + Cup)),
        compiler_params=_PARAMS,
    )(x, w1, w2, wu, bu)


def up_block(sub, skip, w3, b3, w4, b4, wu, bu, *, nb):
    N, h, w, Csub = sub.shape
    _, H, W, Cs = skip.shape
    Cup = wu.shape[-1] // 4
    kern = functools.partial(_up_kernel, h=h, w=w, nb=nb)
    return pl.pallas_call(
        kern,
        out_shape=jax.ShapeDtypeStruct((N, H, W, Cs + Cup), ACT_DTYPE),
        grid=(N // nb,),
        in_specs=[_batch_spec((nb, h, w, Csub)), _batch_spec((nb, H, W, Cs)),
                  _const_spec(w3), _const_spec(b3), _const_spec(w4),
                  _const_spec(b4), _const_spec(wu), _const_spec(bu)],
        out_specs=_batch_spec((nb, H, W, Cs + Cup)),
        compiler_params=_PARAMS,
    )(sub, skip, w3, b3, w4, b4, wu, bu)


def head_block(sub, w3, b3, w4, b4, wf, bf_, *, nb):
    N, H, W, Csub = sub.shape
    nc = wf.shape[-1]
    kern = functools.partial(_head_kernel, h=H, w=W, nb=nb)
    out_spec = _batch_spec((nb, H, W, nc))
    return pl.pallas_call(
        kern,
        out_shape=(jax.ShapeDtypeStruct((N, H, W, nc), jnp.float32),
                   jax.ShapeDtypeStruct((N, H, W, nc), jnp.float32)),
        grid=(N // nb,),
        in_specs=[_batch_spec((nb, H, W, Csub)), _const_spec(w3),
                  _const_spec(b3), _const_spec(w4), _const_spec(b4),
                  _const_spec(wf), _const_spec(bf_)],
        out_specs=(out_spec, out_spec),
        compiler_params=_PARAMS,
    )(sub, w3, b3, w4, b4, wf, bf_)


def kernel(x, o_c1w, o_c1b, o_c2w, o_c2b, o_c3w, o_c3b, o_c4w, o_c4b,
           o_fw, o_fb,
           d3_c1w, d3_c1b, d3_c2w, d3_c2b, d3_c3w, d3_c3b, d3_c4w, d3_c4b,
           d3_uw, d3_ub,
           d2_c1w, d2_c1b, d2_c2w, d2_c2b, d2_c3w, d2_c3b, d2_c4w, d2_c4b,
           d2_uw, d2_ub,
           d1_c1w, d1_c1b, d1_c2w, d1_c2b, d1_c3w, d1_c3b, d1_c4w, d1_c4b,
           d1_uw, d1_ub,
           m_c1w, m_c1b, m_c2w, m_c2b, m_uw, m_ub):
    xh = jnp.transpose(x, (0, 2, 3, 1)).astype(ACT_DTYPE)
    m_uw, m_ub = _pack_uw(m_uw, m_ub)
    d1_uw, d1_ub = _pack_uw(d1_uw, d1_ub)
    d2_uw, d2_ub = _pack_uw(d2_uw, d2_ub)
    d3_uw, d3_ub = _pack_uw(d3_uw, d3_ub)
    (o_c1w, o_c2w, o_c3w, o_c4w, d3_c1w, d3_c2w, d3_c3w, d3_c4w,
     d2_c1w, d2_c2w, d2_c3w, d2_c4w, d1_c1w, d1_c2w, d1_c3w, d1_c4w,
     m_c1w, m_c2w) = [
        _pack_w3(w) for w in
        (o_c1w, o_c2w, o_c3w, o_c4w, d3_c1w, d3_c2w, d3_c3w, d3_c4w,
         d2_c1w, d2_c2w, d2_c3w, d2_c4w, d1_c1w, d1_c2w, d1_c3w, d1_c4w,
         m_c1w, m_c2w)]

    h0 = down_block(xh, o_c1w, o_c2w, pool=False, nb=1)
    h1 = down_block(h0, d3_c1w, d3_c2w, pool=True, nb=1)
    h2 = down_block(h1, d2_c1w, d2_c2w, pool=True, nb=2)
    h3 = down_block(h2, d1_c1w, d1_c2w, pool=True, nb=4)

    u3 = mid_block(h3, m_c1w, m_c2w, m_uw, m_ub, nb=8)
    u2 = up_block(u3, h2, d1_c3w, d1_c3b, d1_c4w, d1_c4b,
                  d1_uw, d1_ub, nb=4)
    u1 = up_block(u2, h1, d2_c3w, d2_c3b, d2_c4w, d2_c4b,
                  d2_uw, d2_ub, nb=2)
    u0 = up_block(u1, h0, d3_c3w, d3_c3b, d3_c4w, d3_c4b,
                  d3_uw, d3_ub, nb=1)

    ls, sm = head_block(u0, o_c3w, o_c3b, o_c4w, o_c4b, o_fw, o_fb, nb=1)
    return {'log_softmax': jnp.transpose(ls, (0, 3, 1, 2)),
            'softmax': jnp.transpose(sm, (0, 3, 1, 2))}
